# baseline (speedup 1.0000x reference)
import jax
import jax.numpy as jnp
from jax import lax
from jax.experimental import pallas as pl
from jax.experimental.pallas import tpu as pltpu
from jax.experimental.pallas import tpu_sc as plsc

_B, _T = 1024, 200
_N = _B * _T
_DIMS = (16, 16, 8, 32, 8, 16, 8, 16, 8)
_NF = len(_DIMS)

_NC, _NS = 2, 16
_NW = _NC * _NS
_NTOK = _N // _NW
_STEP = 128
_NSTEP = _NTOK // _STEP
_NG = _STEP // 16


def _sc_body(xt_ref, *rest):
    w_hbm = rest[:_NF]
    out_refs = rest[_NF:2 * _NF]
    idx_ref = rest[2 * _NF]
    wv = rest[2 * _NF + 1:3 * _NF + 1]
    rows = (rest[3 * _NF + 1:4 * _NF + 1], rest[4 * _NF + 1:5 * _NF + 1])
    wsem = rest[5 * _NF + 1:5 * _NF + 3]

    wid = lax.axis_index("s") * _NC + lax.axis_index("c")
    base = wid * _NTOK

    for i in range(_NF):
        pltpu.sync_copy(w_hbm[i], wv[i])
    pltpu.sync_copy(xt_ref.at[pl.ds(wid * _NSTEP, _NSTEP), :, :], idx_ref)

    iota = lax.iota(jnp.int32, 16)
    one = jnp.full((16,), 1, jnp.int32)

    def compute(step, b):
        @pl.loop(0, _NG)
        def _grp(g):
            tok16 = iota + g * 16
            for i in range(_NF):
                d_i = _DIMS[i]
                idx16 = idx_ref[step, i, pl.ds(g * 16, 16)]
                gpos = idx16 * d_i
                spos = tok16 * d_i
                for _ in range(d_i):
                    vals = plsc.load_gather(wv[i], [gpos])
                    plsc.store_scatter(rows[b][i], [spos], vals)
                    gpos = gpos + one
                    spos = spos + one
        t0 = base + step * _STEP
        for i in range(_NF):
            d_i = _DIMS[i]
            pltpu.async_copy(rows[b][i],
                             out_refs[i].at[pl.ds(t0 * d_i, _STEP * d_i)],
                             wsem[b])

    def drain(b):
        for i in range(_NF):
            d_i = _DIMS[i]
            pltpu.make_async_copy(
                rows[b][i], out_refs[i].at[pl.ds(base * d_i, _STEP * d_i)],
                wsem[b]).wait()

    compute(0, 0)
    compute(1, 1)

    @pl.loop(2, _NSTEP, step=2)
    def _steps(s):
        for b in range(2):
            drain(b)
            compute(s + b, b)

    for b in range(2):
        drain(b)


@jax.jit
def kernel(x, W_msg, W_act, W_finish, W_effect, W_phase, W_position,
           W_number, W_place, W_attrib):
    Ws = (W_msg, W_act, W_finish, W_effect, W_phase, W_position,
          W_number, W_place, W_attrib)
    xt = x.reshape(_N // _STEP, _STEP, _NF).transpose(0, 2, 1)
    Ws = tuple(w.reshape(-1) for w in Ws)

    mesh = plsc.VectorSubcoreMesh(core_axis_name="c", subcore_axis_name="s",
                                  num_cores=_NC, num_subcores=_NS)
    out_type = [jax.ShapeDtypeStruct((_N * d,), jnp.float32) for d in _DIMS]
    scratch = ([pltpu.VMEM((_NSTEP, _NF, _STEP), jnp.int32)]
               + [pltpu.VMEM((v * d,), jnp.float32)
                  for v, d in zip((30, 10, 3, 256, 4, 9, 13, 31, 10), _DIMS)]
               + [pltpu.VMEM((_STEP * d,), jnp.float32) for d in _DIMS]
               + [pltpu.VMEM((_STEP * d,), jnp.float32) for d in _DIMS]
               + [pltpu.SemaphoreType.DMA, pltpu.SemaphoreType.DMA])
    outs = pl.kernel(
        _sc_body,
        out_type=out_type,
        mesh=mesh,
        scratch_types=scratch,
        compiler_params=pltpu.CompilerParams(use_tc_tiling_on_sc=False,
                                             needs_layout_passes=False),
    )(xt, *Ws)
    return tuple(o.reshape(_B, _T, d) for o, d in zip(outs, _DIMS))


# speedup vs baseline: 4.1296x; 4.1296x over previous
import jax
import jax.numpy as jnp
from jax import lax
from jax.experimental import pallas as pl
from jax.experimental.pallas import tpu as pltpu
from jax.experimental.pallas import tpu_sc as plsc

_B, _T = 1024, 200
_DIMS = (16, 16, 8, 32, 8, 16, 8, 16, 8)
_VOCABS = (30, 10, 3, 256, 4, 9, 13, 31, 10)
_NF = len(_DIMS)
_NC, _NS = 2, 16
_DMAX = max(_DIMS)

_COLS = None
_PIPE = 4


def _sc_body(xp_ref, *rest):
    w_hbm = rest[:_NF]
    out_refs = rest[_NF:2 * _NF]
    ib = rest[2 * _NF:2 * _NF + 2]
    ob = rest[2 * _NF + 2:2 * _NF + 4]
    wv = rest[2 * _NF + 4:3 * _NF + 4]
    isem = rest[3 * _NF + 4]
    wsem = rest[3 * _NF + 5:3 * _NF + 7]

    wid = lax.axis_index("s") * _NC + lax.axis_index("c")
    cols = [jnp.full((16,), k, jnp.int32) for k in range(_DMAX)]

    for i in range(_NF):
        pltpu.sync_copy(w_hbm[i], wv[i])

    t_a = wid * 6
    tt0 = t_a >> 3

    def unit(i, ibuf, ob_ref, tile, sub):
        d = _DIMS[i]

        @pl.loop(0, 64)
        def _bj(bj):
            bt = bj >> 3
            j = bj & 7
            idx16 = ibuf[tile, bt, sub, pl.ds(j * 16, 16)]
            pos0 = idx16 * d
            obase = bt * 1024 + j * 16
            pipe = []
            for k in range(d):
                v = plsc.load_gather(wv[i], [pos0 + cols[k]])
                pipe.append((k, v))
                if len(pipe) > _PIPE:
                    kk, vv = pipe.pop(0)
                    ob_ref[pl.ds(obase + (kk >> 3) * 8192 + (kk & 7) * 128,
                                 16)] = vv
            for kk, vv in pipe:
                ob_ref[pl.ds(obase + (kk >> 3) * 8192 + (kk & 7) * 128,
                             16)] = vv

    def post(i, q, t):
        d = _DIMS[i]
        pltpu.async_copy(ob[q].at[pl.ds(0, d * 1024)],
                         out_refs[i].at[pl.ds(t * d * 1024, d * 1024)],
                         wsem[q])

    def drain(i, q):
        d = _DIMS[i]
        pltpu.make_async_copy(ob[q].at[pl.ds(0, d * 1024)],
                              out_refs[i].at[pl.ds(0, d * 1024)],
                              wsem[q]).wait()

    pltpu.sync_copy(xp_ref.at[0, pl.ds(tt0, 2), :, :, :], ib[0])
    for i in range(_NF):
        p = i & 1
        if i < _NF - 1:
            hnext = pltpu.async_copy(
                xp_ref.at[i + 1, pl.ds(tt0, 2), :, :, :], ib[1 - p], isem)
        if i > 0:
            drain(i - 1, 0)
            drain(i - 1, 1)

        d = _DIMS[i]

        @pl.loop(0, 6, step=2)
        def _s(s, i=i, p=p, d=d):
            for q in (0, 1):
                sq = s + q

                @pl.when(sq >= 2)
                def _():
                    drain(i, q)

                t = t_a + sq
                tloc = t - tt0 * 8
                unit(i, ib[p], ob[q], tloc >> 3, tloc & 7)
                post(i, q, t)

        if i < _NF - 1:
            hnext.wait()
    drain(_NF - 1, 0)
    drain(_NF - 1, 1)

    sub_b = wid & 7
    t_b = 192 + sub_b
    for i in range(_NF):
        q = i & 1
        pltpu.sync_copy(xp_ref.at[i, pl.ds(24, 1), :, :, :],
                        ib[1].at[pl.ds(0, 1), :, :, :])
        if i >= 2:
            drain(i - 2, q)
        unit(i, ib[1], ob[q], 0, sub_b)
        post(i, q, t_b)
    drain(_NF - 2, 1)
    drain(_NF - 1, 0)


@jax.jit
def kernel(x, W_msg, W_act, W_finish, W_effect, W_phase, W_position,
           W_number, W_place, W_attrib):
    Ws = (W_msg, W_act, W_finish, W_effect, W_phase, W_position,
          W_number, W_place, W_attrib)
    xp = x.reshape(8, 128, 25, 8, _NF).transpose(4, 2, 0, 3, 1)
    Ws = tuple(w.reshape(-1) for w in Ws)

    mesh = plsc.VectorSubcoreMesh(core_axis_name="c", subcore_axis_name="s",
                                  num_cores=_NC, num_subcores=_NS)
    out_type = [jax.ShapeDtypeStruct((_T * d * 1024,), jnp.float32)
                for d in _DIMS]
    scratch = ([pltpu.VMEM((2, 8, 8, 128), jnp.int32)] * 2
               + [pltpu.VMEM((_DMAX * 1024,), jnp.float32)] * 2
               + [pltpu.VMEM((v * d,), jnp.float32)
                  for v, d in zip(_VOCABS, _DIMS)]
               + [pltpu.SemaphoreType.DMA] * 3)
    outs = pl.kernel(
        _sc_body,
        out_type=out_type,
        mesh=mesh,
        scratch_types=scratch,
        compiler_params=pltpu.CompilerParams(use_tc_tiling_on_sc=False,
                                             needs_layout_passes=False),
    )(xp, *Ws)
    return tuple(
        o.reshape(_T, d // 8, 8, 8, 128).transpose(2, 4, 0, 1, 3)
        .reshape(_B, _T, d)
        for o, d in zip(outs, _DIMS))


# speedup vs baseline: 5.1683x; 1.2515x over previous
import jax
import jax.numpy as jnp
from jax import lax
from jax.experimental import pallas as pl
from jax.experimental.pallas import tpu as pltpu
from jax.experimental.pallas import tpu_sc as plsc

_B, _T = 1024, 200
_DIMS = (16, 16, 8, 32, 8, 16, 8, 16, 8)
_VOCABS = (30, 10, 3, 256, 4, 9, 13, 31, 10)
_NF = len(_DIMS)
_NC, _NS = 2, 16
_EFF = 3
_PIPE = 4

_HALVES = tuple(((0, d),) if d <= 16 else ((0, 16), (16, 16))
                for d in _DIMS)


def _sc_body(xp_ref, *rest):
    w_hbm = rest[:_NF]
    out_refs = rest[_NF:2 * _NF]
    ib = rest[2 * _NF:2 * _NF + 2]
    ob = rest[2 * _NF + 2:2 * _NF + 4]
    wv = rest[2 * _NF + 4:3 * _NF + 4]
    isem = rest[3 * _NF + 4]
    wsem = rest[3 * _NF + 5:3 * _NF + 7]

    wid = lax.axis_index("s") * _NC + lax.axis_index("c")
    iota = lax.iota(jnp.int32, 16)
    cols = [jnp.full((16,), k, jnp.int32) for k in range(32)]
    cols16 = [jnp.full((16,), k * 16, jnp.int32) for k in range(16)]

    for i in range(_NF):
        pltpu.sync_copy(w_hbm[i], wv[i])

    t_a = wid * 6
    tt0 = t_a >> 3

    def unit(i, c0, nc, ibuf, ob_ref, tile, sub):
        d = _DIMS[i]

        @pl.loop(0, 64)
        def _bj(bj):
            bt = bj >> 3
            j = bj & 7
            idx16 = ibuf[tile, bt, sub, pl.ds(j * 16, 16)]
            if i == _EFF:
                pos = idx16 * d + cols[c0]
                kcols = cols
            else:
                pos = idx16 * (d * 16) + iota + cols16[c0]
                kcols = cols16
            obase = bt * 1024 + j * 16
            pipe = []
            for k in range(nc):
                v = plsc.load_gather(wv[i], [pos + kcols[k]])
                pipe.append((k, v))
                if len(pipe) > _PIPE:
                    kk, vv = pipe.pop(0)
                    ob_ref[pl.ds(obase + (kk >> 3) * 8192 + (kk & 7) * 128,
                                 16)] = vv
            for kk, vv in pipe:
                ob_ref[pl.ds(obase + (kk >> 3) * 8192 + (kk & 7) * 128,
                             16)] = vv

    def post(i, h, q, t):
        c0, nc = _HALVES[i][h]
        d = _DIMS[i]
        pltpu.async_copy(ob[q].at[pl.ds(0, nc * 1024)],
                         out_refs[i].at[pl.ds(t * d * 1024 + c0 * 1024,
                                              nc * 1024)],
                         wsem[q])

    def drain(i, h, q):
        nc = _HALVES[i][h][1]
        pltpu.make_async_copy(ob[q].at[pl.ds(0, nc * 1024)],
                              out_refs[i].at[pl.ds(0, nc * 1024)],
                              wsem[q]).wait()

    pltpu.sync_copy(xp_ref.at[0, pl.ds(tt0, 2), :, :, :], ib[0])
    for i in range(_NF):
        p = i & 1
        if i < _NF - 1:
            hnext = pltpu.async_copy(
                xp_ref.at[i + 1, pl.ds(tt0, 2), :, :, :], ib[1 - p], isem)
        if i > 0:
            lh = len(_HALVES[i - 1]) - 1
            drain(i - 1, lh, 0)
            drain(i - 1, lh, 1)

        if len(_HALVES[i]) == 1:
            @pl.loop(0, 6, step=2)
            def _s(s, i=i, p=p):
                for q in (0, 1):
                    sq = s + q

                    @pl.when(sq >= 2)
                    def _():
                        drain(i, 0, q)

                    t = t_a + sq
                    tloc = t - tt0 * 8
                    unit(i, 0, _DIMS[i], ib[p], ob[q], tloc >> 3, tloc & 7)
                    post(i, 0, q, t)
        else:
            @pl.loop(0, 6)
            def _s(s, i=i, p=p):
                @pl.when(s >= 1)
                def _():
                    drain(i, 0, 0)
                    drain(i, 1, 1)

                t = t_a + s
                tloc = t - tt0 * 8
                for h in (0, 1):
                    c0, nc = _HALVES[i][h]
                    unit(i, c0, nc, ib[p], ob[h], tloc >> 3, tloc & 7)
                    post(i, h, h, t)

        if i < _NF - 1:
            hnext.wait()
    lh = len(_HALVES[_NF - 1]) - 1
    drain(_NF - 1, lh, 0)
    drain(_NF - 1, lh, 1)

    sub_b = wid & 7
    t_b = 192 + sub_b
    posted = []
    for i in range(_NF):
        pltpu.sync_copy(xp_ref.at[i, pl.ds(24, 1), :, :, :],
                        ib[1].at[pl.ds(0, 1), :, :, :])
        for h in range(len(_HALVES[i])):
            q = len(posted) & 1
            if len(posted) >= 2:
                pi, ph, pq = posted[len(posted) - 2]
                drain(pi, ph, pq)
            c0, nc = _HALVES[i][h]
            unit(i, c0, nc, ib[1], ob[q], 0, sub_b)
            post(i, h, q, t_b)
            posted.append((i, h, q))
    for pi, ph, pq in posted[-2:]:
        drain(pi, ph, pq)


@jax.jit
def kernel(x, W_msg, W_act, W_finish, W_effect, W_phase, W_position,
           W_number, W_place, W_attrib):
    Ws = (W_msg, W_act, W_finish, W_effect, W_phase, W_position,
          W_number, W_place, W_attrib)
    xp = x.reshape(8, 128, 25, 8, _NF).transpose(4, 2, 0, 3, 1)
    Ws = tuple(
        w.reshape(-1) if i == _EFF else
        jnp.broadcast_to(w.reshape(-1, 1), (w.size, 16)).reshape(-1)
        for i, w in enumerate(Ws))

    mesh = plsc.VectorSubcoreMesh(core_axis_name="c", subcore_axis_name="s",
                                  num_cores=_NC, num_subcores=_NS)
    out_type = [jax.ShapeDtypeStruct((_T * d * 1024,), jnp.float32)
                for d in _DIMS]
    scratch = ([pltpu.VMEM((2, 8, 8, 128), jnp.int32)] * 2
               + [pltpu.VMEM((16 * 1024,), jnp.float32)] * 2
               + [pltpu.VMEM((v * d * (1 if i == _EFF else 16),), jnp.float32)
                  for i, (v, d) in enumerate(zip(_VOCABS, _DIMS))]
               + [pltpu.SemaphoreType.DMA] * 3)
    outs = pl.kernel(
        _sc_body,
        out_type=out_type,
        mesh=mesh,
        scratch_types=scratch,
        compiler_params=pltpu.CompilerParams(use_tc_tiling_on_sc=False,
                                             needs_layout_passes=False),
    )(xp, *Ws)
    return tuple(
        o.reshape(_T, d // 8, 8, 8, 128).transpose(2, 4, 0, 1, 3)
        .reshape(_B, _T, d)
        for o, d in zip(outs, _DIMS))


# speedup vs baseline: 5.3555x; 1.0362x over previous
import jax
import jax.numpy as jnp
from jax import lax
from jax.experimental import pallas as pl
from jax.experimental.pallas import tpu as pltpu
from jax.experimental.pallas import tpu_sc as plsc

_B, _T = 1024, 200
_DIMS = (16, 16, 8, 32, 8, 16, 8, 16, 8)
_VOCABS = (30, 10, 3, 256, 4, 9, 13, 31, 10)
_NF = len(_DIMS)
_NC, _NS = 2, 16
_EFF = 3
_PIPE = 8

_HALVES = tuple(((0, d),) if d <= 16 else ((0, 16), (16, 16))
                for d in _DIMS)


def _sc_body(xp_ref, *rest):
    w_hbm = rest[:_NF]
    out_refs = rest[_NF:2 * _NF]
    ib = rest[2 * _NF:2 * _NF + 2]
    ob = rest[2 * _NF + 2:2 * _NF + 4]
    wv = rest[2 * _NF + 4:3 * _NF + 4]
    isem = rest[3 * _NF + 4]
    wsem = rest[3 * _NF + 5:3 * _NF + 7]

    wid = lax.axis_index("s") * _NC + lax.axis_index("c")
    iota = lax.iota(jnp.int32, 16)
    cols = [jnp.full((16,), k, jnp.int32) for k in range(32)]
    cols16 = [jnp.full((16,), k * 16, jnp.int32) for k in range(16)]

    for i in range(_NF):
        pltpu.sync_copy(w_hbm[i], wv[i])

    t_a = wid * 6
    tt0 = t_a >> 3

    def unit(i, c0, nc, ibuf, ob_ref, tile, sub):
        d = _DIMS[i]

        @pl.loop(0, 64)
        def _bj(bj):
            bt = bj >> 3
            j = bj & 7
            idx16 = ibuf[tile, bt, sub, pl.ds(j * 16, 16)]
            if i == _EFF:
                pos = idx16 * d + cols[c0]
                kcols = cols
            else:
                pos = idx16 * (d * 16) + iota + cols16[c0]
                kcols = cols16
            obase = bt * 1024 + j * 16
            pipe = []
            for k in range(nc):
                v = plsc.load_gather(wv[i], [pos + kcols[k]])
                pipe.append((k, v))
                if len(pipe) > _PIPE:
                    kk, vv = pipe.pop(0)
                    ob_ref[pl.ds(obase + (kk >> 3) * 8192 + (kk & 7) * 128,
                                 16)] = vv
            for kk, vv in pipe:
                ob_ref[pl.ds(obase + (kk >> 3) * 8192 + (kk & 7) * 128,
                             16)] = vv

    def post(i, h, q, t):
        c0, nc = _HALVES[i][h]
        d = _DIMS[i]
        pltpu.async_copy(ob[q].at[pl.ds(0, nc * 1024)],
                         out_refs[i].at[pl.ds(t * d * 1024 + c0 * 1024,
                                              nc * 1024)],
                         wsem[q])

    def drain(i, h, q):
        nc = _HALVES[i][h][1]
        pltpu.make_async_copy(ob[q].at[pl.ds(0, nc * 1024)],
                              out_refs[i].at[pl.ds(0, nc * 1024)],
                              wsem[q]).wait()

    pltpu.sync_copy(xp_ref.at[0, pl.ds(tt0, 2), :, :, :], ib[0])
    for i in range(_NF):
        p = i & 1
        if i < _NF - 1:
            hnext = pltpu.async_copy(
                xp_ref.at[i + 1, pl.ds(tt0, 2), :, :, :], ib[1 - p], isem)
        if i > 0:
            lh = len(_HALVES[i - 1]) - 1
            drain(i - 1, lh, 0)
            drain(i - 1, lh, 1)

        if len(_HALVES[i]) == 1:
            @pl.loop(0, 6, step=2)
            def _s(s, i=i, p=p):
                for q in (0, 1):
                    sq = s + q

                    @pl.when(sq >= 2)
                    def _():
                        drain(i, 0, q)

                    t = t_a + sq
                    tloc = t - tt0 * 8
                    unit(i, 0, _DIMS[i], ib[p], ob[q], tloc >> 3, tloc & 7)
                    post(i, 0, q, t)
        else:
            @pl.loop(0, 6)
            def _s(s, i=i, p=p):
                @pl.when(s >= 1)
                def _():
                    drain(i, 0, 0)
                    drain(i, 1, 1)

                t = t_a + s
                tloc = t - tt0 * 8
                for h in (0, 1):
                    c0, nc = _HALVES[i][h]
                    unit(i, c0, nc, ib[p], ob[h], tloc >> 3, tloc & 7)
                    post(i, h, h, t)

        if i < _NF - 1:
            hnext.wait()
    lh = len(_HALVES[_NF - 1]) - 1
    drain(_NF - 1, lh, 0)
    drain(_NF - 1, lh, 1)

    sub_b = wid & 7
    t_b = 192 + sub_b
    posted = []
    for i in range(_NF):
        pltpu.sync_copy(xp_ref.at[i, pl.ds(24, 1), :, :, :],
                        ib[1].at[pl.ds(0, 1), :, :, :])
        for h in range(len(_HALVES[i])):
            q = len(posted) & 1
            if len(posted) >= 2:
                pi, ph, pq = posted[len(posted) - 2]
                drain(pi, ph, pq)
            c0, nc = _HALVES[i][h]
            unit(i, c0, nc, ib[1], ob[q], 0, sub_b)
            post(i, h, q, t_b)
            posted.append((i, h, q))
    for pi, ph, pq in posted[-2:]:
        drain(pi, ph, pq)


@jax.jit
def kernel(x, W_msg, W_act, W_finish, W_effect, W_phase, W_position,
           W_number, W_place, W_attrib):
    Ws = (W_msg, W_act, W_finish, W_effect, W_phase, W_position,
          W_number, W_place, W_attrib)
    xp = x.reshape(8, 128, 25, 8, _NF).transpose(4, 2, 0, 3, 1)
    Ws = tuple(
        w.reshape(-1) if i == _EFF else
        jnp.broadcast_to(w.reshape(-1, 1), (w.size, 16)).reshape(-1)
        for i, w in enumerate(Ws))

    mesh = plsc.VectorSubcoreMesh(core_axis_name="c", subcore_axis_name="s",
                                  num_cores=_NC, num_subcores=_NS)
    out_type = [jax.ShapeDtypeStruct((_T * d * 1024,), jnp.float32)
                for d in _DIMS]
    scratch = ([pltpu.VMEM((2, 8, 8, 128), jnp.int32)] * 2
               + [pltpu.VMEM((16 * 1024,), jnp.float32)] * 2
               + [pltpu.VMEM((v * d * (1 if i == _EFF else 16),), jnp.float32)
                  for i, (v, d) in enumerate(zip(_VOCABS, _DIMS))]
               + [pltpu.SemaphoreType.DMA] * 3)
    outs = pl.kernel(
        _sc_body,
        out_type=out_type,
        mesh=mesh,
        scratch_types=scratch,
        compiler_params=pltpu.CompilerParams(use_tc_tiling_on_sc=False,
                                             needs_layout_passes=False),
    )(xp, *Ws)
    return tuple(
        o.reshape(_T, d // 8, 8, 8, 128).transpose(2, 4, 0, 1, 3)
        .reshape(_B, _T, d)
        for o, d in zip(outs, _DIMS))


# speedup vs baseline: 5.5402x; 1.0345x over previous
import jax
import jax.numpy as jnp
from jax import lax
from jax.experimental import pallas as pl
from jax.experimental.pallas import tpu as pltpu
from jax.experimental.pallas import tpu_sc as plsc

_B, _T = 1024, 200
_DIMS = (16, 16, 8, 32, 8, 16, 8, 16, 8)
_VOCABS = (30, 10, 3, 256, 4, 9, 13, 31, 10)
_NF = len(_DIMS)
_NC, _NS = 2, 16
_EFF = 3
_PIPE = 8

_HALVES = tuple(((0, d),) if d <= 16 else ((0, 16), (16, 16))
                for d in _DIMS)


def _sc_body(xp_ref, *rest):
    w_hbm = rest[:_NF]
    out_refs = rest[_NF:2 * _NF]
    ib = rest[2 * _NF:2 * _NF + 2]
    ob = rest[2 * _NF + 2:2 * _NF + 4]
    wv = rest[2 * _NF + 4:3 * _NF + 4]
    isem = rest[3 * _NF + 4]
    wsem = rest[3 * _NF + 5:3 * _NF + 7]

    wid = lax.axis_index("s") * _NC + lax.axis_index("c")
    iota = lax.iota(jnp.int32, 16)
    cols = [jnp.full((16,), k, jnp.int32) for k in range(32)]
    cols16 = [jnp.full((16,), k * 16, jnp.int32) for k in range(16)]

    for i in range(_NF):
        pltpu.sync_copy(w_hbm[i], wv[i])

    t_a = wid * 6
    tt0 = t_a >> 3

    def unit(i, c0, nc, ibuf, ob_ref, tile, sub):
        d = _DIMS[i]

        @pl.loop(0, 64)
        def _bj(bj):
            bt = bj >> 3
            j = bj & 7
            idx16 = ibuf[tile, bt, sub, pl.ds(j * 16, 16)]
            if i == _EFF:
                pos = idx16 * d + cols[c0]
                kcols = cols
            else:
                pos = idx16 * (d * 16) + iota + cols16[c0]
                kcols = cols16
            obase = bt * 1024 + j * 16
            pipe = []
            for k in range(nc):
                v = plsc.load_gather(wv[i], [pos + kcols[k]])
                pipe.append((k, v))
                if len(pipe) > _PIPE:
                    kk, vv = pipe.pop(0)
                    ob_ref[pl.ds(obase + (kk >> 3) * 8192 + (kk & 7) * 128,
                                 16)] = vv
            for kk, vv in pipe:
                ob_ref[pl.ds(obase + (kk >> 3) * 8192 + (kk & 7) * 128,
                             16)] = vv

    def post(i, h, q, t):
        c0, nc = _HALVES[i][h]
        d = _DIMS[i]
        pltpu.async_copy(ob[q].at[pl.ds(0, nc * 1024)],
                         out_refs[i].at[pl.ds(t * d * 1024 + c0 * 1024,
                                              nc * 1024)],
                         wsem[q])

    def drain(i, h, q):
        nc = _HALVES[i][h][1]
        pltpu.make_async_copy(ob[q].at[pl.ds(0, nc * 1024)],
                              out_refs[i].at[pl.ds(0, nc * 1024)],
                              wsem[q]).wait()

    pltpu.sync_copy(xp_ref.at[0, pl.ds(tt0, 2), :, :, :], ib[0])
    for i in range(_NF):
        p = i & 1
        if i < _NF - 1:
            hnext = pltpu.async_copy(
                xp_ref.at[i + 1, pl.ds(tt0, 2), :, :, :], ib[1 - p], isem)
        else:
            pltpu.async_copy(xp_ref.at[0, pl.ds(24, 1), :, :, :],
                             ib[1].at[pl.ds(0, 1), :, :, :], isem)
        if i > 0:
            lh = len(_HALVES[i - 1]) - 1
            drain(i - 1, lh, 0)
            drain(i - 1, lh, 1)

        if len(_HALVES[i]) == 1:
            @pl.loop(0, 6, step=2)
            def _s(s, i=i, p=p):
                for q in (0, 1):
                    sq = s + q

                    @pl.when(sq >= 2)
                    def _():
                        drain(i, 0, q)

                    t = t_a + sq
                    tloc = t - tt0 * 8
                    unit(i, 0, _DIMS[i], ib[p], ob[q], tloc >> 3, tloc & 7)
                    post(i, 0, q, t)
        else:
            @pl.loop(0, 6)
            def _s(s, i=i, p=p):
                @pl.when(s >= 1)
                def _():
                    drain(i, 0, 0)
                    drain(i, 1, 1)

                t = t_a + s
                tloc = t - tt0 * 8
                for h in (0, 1):
                    c0, nc = _HALVES[i][h]
                    unit(i, c0, nc, ib[p], ob[h], tloc >> 3, tloc & 7)
                    post(i, h, h, t)

        if i < _NF - 1:
            hnext.wait()
    lh = len(_HALVES[_NF - 1]) - 1
    drain(_NF - 1, lh, 0)
    drain(_NF - 1, lh, 1)

    sub_b = wid & 7
    t_b = 192 + sub_b
    posted = []
    for i in range(_NF):
        pb = (i + 1) & 1
        pltpu.make_async_copy(xp_ref.at[i, pl.ds(24, 1), :, :, :],
                              ib[pb].at[pl.ds(0, 1), :, :, :], isem).wait()
        if i < _NF - 1:
            pltpu.async_copy(xp_ref.at[i + 1, pl.ds(24, 1), :, :, :],
                             ib[(i + 2) & 1].at[pl.ds(0, 1), :, :, :], isem)
        for h in range(len(_HALVES[i])):
            q = len(posted) & 1
            if len(posted) >= 2:
                pi, ph, pq = posted[len(posted) - 2]
                drain(pi, ph, pq)
            c0, nc = _HALVES[i][h]
            unit(i, c0, nc, ib[pb], ob[q], 0, sub_b)
            post(i, h, q, t_b)
            posted.append((i, h, q))
    for pi, ph, pq in posted[-2:]:
        drain(pi, ph, pq)


@jax.jit
def kernel(x, W_msg, W_act, W_finish, W_effect, W_phase, W_position,
           W_number, W_place, W_attrib):
    Ws = (W_msg, W_act, W_finish, W_effect, W_phase, W_position,
          W_number, W_place, W_attrib)
    xp = x.reshape(8, 128, 25, 8, _NF).transpose(4, 2, 0, 3, 1)
    Ws = tuple(
        w.reshape(-1) if i == _EFF else
        jnp.broadcast_to(w.reshape(-1, 1), (w.size, 16)).reshape(-1)
        for i, w in enumerate(Ws))

    mesh = plsc.VectorSubcoreMesh(core_axis_name="c", subcore_axis_name="s",
                                  num_cores=_NC, num_subcores=_NS)
    out_type = [jax.ShapeDtypeStruct((_T * d * 1024,), jnp.float32)
                for d in _DIMS]
    scratch = ([pltpu.VMEM((2, 8, 8, 128), jnp.int32)] * 2
               + [pltpu.VMEM((16 * 1024,), jnp.float32)] * 2
               + [pltpu.VMEM((v * d * (1 if i == _EFF else 16),), jnp.float32)
                  for i, (v, d) in enumerate(zip(_VOCABS, _DIMS))]
               + [pltpu.SemaphoreType.DMA] * 3)
    outs = pl.kernel(
        _sc_body,
        out_type=out_type,
        mesh=mesh,
        scratch_types=scratch,
        compiler_params=pltpu.CompilerParams(use_tc_tiling_on_sc=False,
                                             needs_layout_passes=False),
    )(xp, *Ws)
    return tuple(
        o.reshape(_T, d // 8, 8, 8, 128).transpose(2, 4, 0, 1, 3)
        .reshape(_B, _T, d)
        for o, d in zip(outs, _DIMS))
